# baseline (speedup 1.0000x reference)
import functools

import jax
import jax.numpy as jnp
from jax import lax
from jax.experimental import pallas as pl
from jax.experimental.pallas import tpu as pltpu
from jax.experimental.pallas import tpu_sc as plsc

DIM = 64
EPS = 1e-12
LANES = 16
NC = 2
NS = 16
NW = NC * NS
CHUNK = 512
DMA_SPLIT = 128


_GATHER_DNUMS = lax.GatherDimensionNumbers(
    offset_dims=(), collapsed_slice_dims=(0,), start_index_map=(0,)
)


def _shuffle(v, idx):
    return lax.gather(
        v, idx[:, None], _GATHER_DNUMS, slice_sizes=(1,),
        mode=lax.GatherScatterMode.PROMISE_IN_BOUNDS,
    )


def _allsum(v):
    for sh in (1, 2, 4, 8):
        idx = jnp.arange(LANES, dtype=jnp.int32) ^ sh
        v = v + _shuffle(v, idx)
    return v


def _rsqrt(v):
    i = plsc.bitcast(v, jnp.int32)
    i = jnp.int32(0x5F3759DF) - (i >> 1)
    y = plsc.bitcast(i, jnp.float32)
    hv = 0.5 * v
    for _ in range(3):
        y = y * (1.5 - hv * y * y)
    return y


def kernel(input_tokens, table, oov, ln_weight, ln_bias):
    n, m = input_tokens.shape
    b = n * m
    tok = input_tokens.reshape(b).astype(jnp.int32)
    oov_flat = oov.reshape(DIM)

    rows_w = b // NW
    n_chunks = rows_w // CHUNK
    mesh = plsc.VectorSubcoreMesh(
        core_axis_name="c", subcore_axis_name="s", num_cores=NC, num_subcores=NS
    )

    @functools.partial(
        pl.kernel,
        mesh=mesh,
        compiler_params=pltpu.CompilerParams(
            needs_layout_passes=False, use_tc_tiling_on_sc=False
        ),
        out_type=jax.ShapeDtypeStruct((b, DIM), jnp.float32),
        scratch_types=[
            pltpu.VMEM((CHUNK,), jnp.int32),
            pltpu.VMEM((CHUNK, DIM), jnp.float32),
            pltpu.VMEM((DIM,), jnp.float32),
            pltpu.VMEM((DIM,), jnp.float32),
            pltpu.VMEM((DIM,), jnp.float32),
            pltpu.SemaphoreType.DMA,
        ],
    )
    def run(tok_hbm, table_hbm, oov_hbm, w_hbm, b_hbm, out_hbm,
            idx_v, rows_v, oov_v, w_v, b_v, sem):
        wid = lax.axis_index("s") * NC + lax.axis_index("c")
        base = wid * rows_w

        pltpu.sync_copy(oov_hbm, oov_v)
        pltpu.sync_copy(w_hbm, w_v)
        pltpu.sync_copy(b_hbm, b_v)

        oov_r = [oov_v[pl.ds(16 * j, 16)] for j in range(4)]
        w_r = [w_v[pl.ds(16 * j, 16)] for j in range(4)]
        b_r = [b_v[pl.ds(16 * j, 16)] for j in range(4)]

        def chunk_body(g, carry):
            cbase = base + g * CHUNK
            pltpu.sync_copy(tok_hbm.at[pl.ds(cbase, CHUNK)], idx_v)
            copies = [
                pltpu.async_copy(
                    table_hbm.at[idx_v.at[pl.ds(j * DMA_SPLIT, DMA_SPLIT)]],
                    rows_v.at[pl.ds(j * DMA_SPLIT, DMA_SPLIT)],
                    sem,
                )
                for j in range(CHUNK // DMA_SPLIT)
            ]
            for cp in copies:
                cp.wait()

            def grp_body(t, carry2):
                r0 = t * LANES
                tokv = idx_v[pl.ds(r0, LANES)]
                maskv = jnp.where(tokv == 1, 1.0, 0.0).astype(jnp.float32)
                for lane in range(LANES):
                    r = r0 + lane
                    mask = _shuffle(maskv, jnp.full((LANES,), lane, jnp.int32))
                    keep = 1.0 - mask
                    xs = []
                    for j in range(4):
                        g_j = rows_v[r, pl.ds(16 * j, 16)]
                        xs.append(g_j * keep + oov_r[j] * mask)
                    total = _allsum(xs[0] + xs[1] + xs[2] + xs[3])
                    mean = total * (1.0 / DIM)
                    sq = _allsum(xs[0] * xs[0] + xs[1] * xs[1]
                                 + xs[2] * xs[2] + xs[3] * xs[3])
                    var = sq * (1.0 / DIM) - mean * mean + EPS
                    inv = _rsqrt(var)
                    for j in range(4):
                        y = (xs[j] - mean) * inv
                        rows_v[r, pl.ds(16 * j, 16)] = y * w_r[j] + b_r[j]
                return carry2

            lax.fori_loop(0, CHUNK // LANES, grp_body, 0)
            pltpu.sync_copy(rows_v, out_hbm.at[pl.ds(cbase, CHUNK)])
            return carry

        lax.fori_loop(0, n_chunks, chunk_body, 0)

    out = run(tok, table, oov_flat, ln_weight, ln_bias)
    return out.reshape(n, m, DIM)
